# Initial kernel scaffold; baseline (speedup 1.0000x reference)
import jax
import jax.numpy as jnp
from jax.experimental import pallas as pl


def kernel(edge_index, hop, remaining_distance, expected_travel_time, current_time, curr2neigh, curr2target, nid, curr_node, aW2, ab2, aW3, ab3, aW4, ab4, cW2, cb2, cW3, cb3, cW4, cb4):
    raise NotImplementedError("write your pallas kernel here")


# speedup vs baseline: 60.6883x; 60.6883x over previous
import functools

import jax
import jax.numpy as jnp
from jax import lax
from jax.experimental import pallas as pl
from jax.experimental.pallas import tpu as pltpu
from jax.experimental.pallas import tpu_sc as plsc

N = 100000
E = 6400000
BLK = 2048
NBLK = 49
NP = BLK * NBLK

NC = 2
NS = 16
NW = NC * NS
EPW = E // NW
CH = 2000
NCHUNK = EPW // CH
W = 8
RPT = NP // NS
SRPT = RPT // 8


def _prep_body(hop, rd, ett, ct, c2n, c2t, tab_ref, pmax_ref, psum_ref):
    b = pl.program_id(0)
    hopv = hop[...]
    rdv = rd[...]
    ettv = ett[...]
    ctv = ct[...] * (1.0 / 1440.0)
    a = c2n[...]
    t = c2t[...]
    na = jnp.sqrt(jnp.sum(a * a, axis=1, keepdims=True))
    nb = jnp.sqrt(jnp.sum(t * t, axis=1, keepdims=True))
    cos = jnp.sum(a * t, axis=1, keepdims=True) / (
        jnp.maximum(na, 1e-6) * jnp.maximum(nb, 1e-6))
    ridx = b * BLK + lax.broadcasted_iota(jnp.int32, (BLK, 1), 0)
    valid = (ridx < N).astype(jnp.float32)
    zero = jnp.zeros((BLK, 1), jnp.float32)
    tabblk = jnp.concatenate(
        [ctv, hopv, cos, rdv, ettv, valid, zero, zero], axis=1)
    tab_ref[...] = tabblk
    pmax_ref[0] = jnp.max(jnp.abs(tabblk), axis=0, keepdims=True)
    psum_ref[0] = jnp.sum(tabblk, axis=0, keepdims=True)


def _prep(hop, rd, ett, ct, c2n, c2t):
    return pl.pallas_call(
        _prep_body,
        grid=(NBLK,),
        in_specs=[
            pl.BlockSpec((BLK, 1), lambda b: (b, 0)),
            pl.BlockSpec((BLK, 1), lambda b: (b, 0)),
            pl.BlockSpec((BLK, 1), lambda b: (b, 0)),
            pl.BlockSpec((BLK, 1), lambda b: (b, 0)),
            pl.BlockSpec((BLK, 4), lambda b: (b, 0)),
            pl.BlockSpec((BLK, 4), lambda b: (b, 0)),
        ],
        out_specs=[
            pl.BlockSpec((BLK, W), lambda b: (b, 0)),
            pl.BlockSpec((1, 1, W), lambda b: (b, 0, 0)),
            pl.BlockSpec((1, 1, W), lambda b: (b, 0, 0)),
        ],
        out_shape=[
            jax.ShapeDtypeStruct((NP, W), jnp.float32),
            jax.ShapeDtypeStruct((NBLK, 1, W), jnp.float32),
            jax.ShapeDtypeStruct((NBLK, 1, W), jnp.float32),
        ],
    )(hop, rd, ett, ct, c2n, c2t)


def _segsum_body(tab_hbm, zeros_hbm, src_hbm, dst_hbm, out_hbm,
                 tab_sh, acc_sh, stage_v, src_v, dst_v, rows_v, sem):
    c = lax.axis_index("c")
    s = lax.axis_index("s")
    wid = c * NS + s
    rb = s * RPT

    def stage(j, carry):
        r = rb + j * SRPT
        pltpu.sync_copy(tab_hbm.at[pl.ds(r, SRPT)], stage_v)
        pltpu.sync_copy(stage_v, tab_sh.at[pl.ds(r, SRPT)])
        pltpu.sync_copy(zeros_hbm.at[pl.ds(r, SRPT)], stage_v)
        pltpu.sync_copy(stage_v, acc_sh.at[pl.ds(r, SRPT)])
        return carry

    lax.fori_loop(0, 8, stage, 0)
    plsc.subcore_barrier()

    ebase = wid * EPW

    def body(g, carry):
        off = ebase + g * CH
        pltpu.sync_copy(src_hbm.at[pl.ds(off, CH)], src_v)
        pltpu.sync_copy(dst_hbm.at[pl.ds(off, CH)], dst_v)
        pltpu.async_copy(tab_sh.at[src_v], rows_v, sem).wait()
        pltpu.sync_copy(rows_v, acc_sh.at[dst_v], add=True)
        return carry

    lax.fori_loop(0, NCHUNK, body, 0)
    plsc.subcore_barrier()

    def drain(j, carry):
        r = rb + j * SRPT
        pltpu.sync_copy(acc_sh.at[pl.ds(r, SRPT)], stage_v)
        pltpu.sync_copy(stage_v, out_hbm.at[c].at[pl.ds(r, SRPT)])
        return carry

    lax.fori_loop(0, 8, drain, 0)


_segsum = functools.partial(
    pl.kernel,
    out_type=jax.ShapeDtypeStruct((NC, NP, W), jnp.float32),
    mesh=plsc.VectorSubcoreMesh(core_axis_name="c", subcore_axis_name="s"),
    compiler_params=pltpu.CompilerParams(use_tc_tiling_on_sc=False),
    scratch_types=[
        pltpu.VMEM_SHARED((NP, W), jnp.float32),
        pltpu.VMEM_SHARED((NP, W), jnp.float32),
        pltpu.VMEM((SRPT, W), jnp.float32),
        pltpu.VMEM((CH,), jnp.int32),
        pltpu.VMEM((CH,), jnp.int32),
        pltpu.VMEM((CH, W), jnp.float32),
        pltpu.SemaphoreType.DMA,
    ],
)(_segsum_body)


def _final_body(tab, parts, invm, ssum, cnod,
                aW2, ab2, aW3, ab3, aW4c, ab4, cW2, cb2, cW3, cb3, cW4c, cb4,
                pol_ref, val_ref, lscr, mscr, escr, xscr):
    ph = pl.program_id(0)
    b = pl.program_id(1)

    @pl.when(ph == 0)
    def _mlp():
        @pl.when(b == 0)
        def _init():
            mscr[0, 0] = -1e30
            xscr[...] = jnp.zeros((1, W), jnp.float32)

        acc = parts[0] + parts[1]
        deg = jnp.maximum(acc[:, 5:6], 1.0)
        x8 = acc / deg
        iv = invm[...]
        featn = tab[...] * iv
        xn = x8 * iv
        h = (featn + xn) * 0.5
        a1 = jax.nn.sigmoid(jnp.dot(h, aW2[...],
                                    preferred_element_type=jnp.float32)
                            + ab2[...])
        a2 = jax.nn.sigmoid(jnp.dot(a1, aW3[...],
                                    preferred_element_type=jnp.float32)
                            + ab3[...])
        logit = jnp.sum(a2 * aW4c[...], axis=1, keepdims=True) + ab4[0, 0]
        ridx = b * BLK + lax.broadcasted_iota(jnp.int32, (BLK, 1), 0)
        bad = (ridx >= N) | (ridx == cnod[0, 0])
        logit = jnp.where(bad, -1e30, logit)
        lscr[b, :, :] = logit
        mscr[0, 0] = jnp.maximum(mscr[0, 0], jnp.max(logit))
        xscr[...] = xscr[...] + jnp.sum(x8, axis=0, keepdims=True)

    @pl.when(ph == 1)
    def _exp():
        @pl.when(b == 0)
        def _init():
            escr[0, 0] = 0.0

        e = jnp.exp(lscr[b, :, :] - mscr[0, 0])
        lscr[b, :, :] = e
        escr[0, 0] = escr[0, 0] + jnp.sum(e)

    @pl.when(ph == 2)
    def _out():
        pol_ref[...] = lscr[b, :, :] * (1.0 / escr[0, 0])

        @pl.when(b == NBLK - 1)
        def _critic():
            iv = invm[...]
            s_ = ssum[...] * iv
            xs = xscr[...] * iv
            rn = 1.0 / float(N)
            cc = jnp.concatenate(
                [s_[:, 0:1], s_[:, 1:2], s_[:, 3:4],
                 xs[:, 0:1], xs[:, 1:2], xs[:, 3:4]], axis=1) * rn
            v1 = jax.nn.sigmoid(jnp.dot(cc, cW2[...],
                                        preferred_element_type=jnp.float32)
                                + cb2[...])
            v2 = jax.nn.sigmoid(jnp.dot(v1, cW3[...],
                                        preferred_element_type=jnp.float32)
                                + cb3[...])
            val_ref[0, 0] = jnp.sum(v2 * cW4c[...]) + cb4[0, 0]


def _final(tab, parts, invm, ssum, cnod,
           aW2p, ab2, aW3, ab3, aW4c, ab4, cW2, cb2, cW3, cb3, cW4c, cb4):
    reps = lambda shape: pl.BlockSpec(shape, lambda p, b: tuple(0 for _ in shape))
    return pl.pallas_call(
        _final_body,
        grid=(3, NBLK),
        in_specs=[
            pl.BlockSpec((BLK, W), lambda p, b: (b, 0)),
            pl.BlockSpec((NC, BLK, W), lambda p, b: (0, b, 0)),
            reps((1, W)),
            reps((1, W)),
            pl.BlockSpec(memory_space=pltpu.SMEM),
            reps((W, 256)),
            reps((1, 256)),
            reps((256, 128)),
            reps((1, 128)),
            reps((1, 128)),
            pl.BlockSpec(memory_space=pltpu.SMEM),
            reps((6, 256)),
            reps((1, 256)),
            reps((256, 128)),
            reps((1, 128)),
            reps((1, 128)),
            pl.BlockSpec(memory_space=pltpu.SMEM),
        ],
        out_specs=[
            pl.BlockSpec((BLK, 1), lambda p, b: (b, 0)),
            pl.BlockSpec(memory_space=pltpu.SMEM),
        ],
        out_shape=[
            jax.ShapeDtypeStruct((NP, 1), jnp.float32),
            jax.ShapeDtypeStruct((1, 1), jnp.float32),
        ],
        scratch_shapes=[
            pltpu.VMEM((NBLK, BLK, 1), jnp.float32),
            pltpu.SMEM((1, 1), jnp.float32),
            pltpu.SMEM((1, 1), jnp.float32),
            pltpu.VMEM((1, W), jnp.float32),
        ],
    )(tab, parts, invm, ssum, cnod,
      aW2p, ab2, aW3, ab3, aW4c, ab4, cW2, cb2, cW3, cb3, cW4c, cb4)


def kernel(edge_index, hop, remaining_distance, expected_travel_time,
           current_time, curr2neigh, curr2target, nid, curr_node,
           aW2, ab2, aW3, ab3, aW4, ab4, cW2, cb2, cW3, cb3, cW4, cb4):
    pad = NP - N
    padr = lambda x: jnp.pad(x, ((0, pad), (0, 0)))
    hopP = padr(hop)
    rdP = padr(remaining_distance)
    ettP = padr(expected_travel_time)
    ctP = padr(current_time)
    c2nP = jnp.pad(curr2neigh, ((0, pad), (0, 1)))
    c2tP = jnp.pad(curr2target, ((0, pad), (0, 1)))

    tab, pmax, psum = _prep(hopP, rdP, ettP, ctP, c2nP, c2tP)

    m = jnp.max(pmax[:, 0, :], axis=0)
    ssum = jnp.sum(psum[:, 0, :], axis=0, keepdims=True)
    colw = jnp.arange(W)
    norm_col = (colw >= 1) & (colw <= 4)
    invm = jnp.where(norm_col, 1.0 / jnp.maximum(m, 1e-12), 1.0)
    invm = invm.reshape(1, W).astype(jnp.float32)

    src = edge_index[0]
    dst = edge_index[1]
    zeros = jnp.zeros((NP, W), jnp.float32)
    parts = _segsum(tab, zeros, src, dst)

    aW2p = jnp.pad(aW2, ((0, 3), (0, 0)))
    cnod = jnp.asarray(curr_node, jnp.int32).reshape(1, 1)
    pol_pad, val = _final(
        tab, parts, invm, ssum, cnod,
        aW2p, ab2.reshape(1, 256), aW3, ab3.reshape(1, 128),
        aW4.reshape(1, 128), ab4.reshape(1, 1).astype(jnp.float32),
        cW2, cb2.reshape(1, 256), cW3, cb3.reshape(1, 128),
        cW4.reshape(1, 128), cb4.reshape(1, 1).astype(jnp.float32))

    policy_mask = pol_pad[:N, 0]
    value = val.reshape(())
    return policy_mask, value


# speedup vs baseline: 86.5906x; 1.4268x over previous
import functools

import jax
import jax.numpy as jnp
from jax import lax
from jax.experimental import pallas as pl
from jax.experimental.pallas import tpu as pltpu
from jax.experimental.pallas import tpu_sc as plsc

N = 100000
E = 6400000
BLK = 2048
NBLK = 49
NP = BLK * NBLK

NC = 2
NS = 16
NW = NC * NS
EPW = E // NW
CH = 1000
NCHUNK = EPW // CH
W = 8
RPT = NP // NS
SRPT = RPT // 8


def _prep_body(hop, rd, ett, ct, c2n, c2t, tab_ref, pmax_ref, psum_ref):
    b = pl.program_id(0)
    ridx = b * BLK + lax.broadcasted_iota(jnp.int32, (BLK, 1), 0)
    ok = ridx < N
    hopv = jnp.where(ok, hop[...], 0.0)
    rdv = jnp.where(ok, rd[...], 0.0)
    ettv = jnp.where(ok, ett[...], 0.0)
    ctv = jnp.where(ok, ct[...] * (1.0 / 1440.0), 0.0)
    a = c2n[...]
    t = c2t[...]
    na = jnp.sqrt(jnp.sum(a * a, axis=1, keepdims=True))
    nb = jnp.sqrt(jnp.sum(t * t, axis=1, keepdims=True))
    cos = jnp.sum(a * t, axis=1, keepdims=True) / (
        jnp.maximum(na, 1e-6) * jnp.maximum(nb, 1e-6))
    cos = jnp.where(ok, cos, 0.0)
    valid = ok.astype(jnp.float32)
    zero = jnp.zeros((BLK, 1), jnp.float32)
    tabblk = jnp.concatenate(
        [ctv, hopv, cos, rdv, ettv, valid, zero, zero], axis=1)
    tab_ref[...] = tabblk
    pmax_ref[0] = jnp.max(jnp.abs(tabblk), axis=0, keepdims=True)
    psum_ref[0] = jnp.sum(tabblk, axis=0, keepdims=True)


def _prep(hop, rd, ett, ct, c2n, c2t):
    return pl.pallas_call(
        _prep_body,
        grid=(NBLK,),
        in_specs=[
            pl.BlockSpec((BLK, 1), lambda b: (b, 0)),
            pl.BlockSpec((BLK, 1), lambda b: (b, 0)),
            pl.BlockSpec((BLK, 1), lambda b: (b, 0)),
            pl.BlockSpec((BLK, 1), lambda b: (b, 0)),
            pl.BlockSpec((BLK, 3), lambda b: (b, 0)),
            pl.BlockSpec((BLK, 3), lambda b: (b, 0)),
        ],
        out_specs=[
            pl.BlockSpec((BLK, W), lambda b: (b, 0)),
            pl.BlockSpec((1, 1, W), lambda b: (b, 0, 0)),
            pl.BlockSpec((1, 1, W), lambda b: (b, 0, 0)),
        ],
        out_shape=[
            jax.ShapeDtypeStruct((NP, W), jnp.float32),
            jax.ShapeDtypeStruct((NBLK, 1, W), jnp.float32),
            jax.ShapeDtypeStruct((NBLK, 1, W), jnp.float32),
        ],
    )(hop, rd, ett, ct, c2n, c2t)


def _segsum_body(tab_hbm, zeros_hbm, src_hbm, dst_hbm, out_hbm,
                 tab_sh, acc_sh, stage_v,
                 src0, src1, dst0, dst1, rows0, rows1,
                 isem0, isem1, gsem, ssem0, ssem1):
    c = lax.axis_index("c")
    s = lax.axis_index("s")
    wid = c * NS + s
    rb = s * RPT

    def stage(j, carry):
        r = rb + j * SRPT
        pltpu.sync_copy(tab_hbm.at[pl.ds(r, SRPT)], stage_v)
        pltpu.sync_copy(stage_v, tab_sh.at[pl.ds(r, SRPT)])
        pltpu.sync_copy(zeros_hbm.at[pl.ds(r, SRPT)], stage_v)
        pltpu.sync_copy(stage_v, acc_sh.at[pl.ds(r, SRPT)])
        return carry

    lax.fori_loop(0, 8, stage, 0)
    plsc.subcore_barrier()

    ebase = wid * EPW
    srcs = (src0, src1)
    dsts = (dst0, dst1)
    rows = (rows0, rows1)
    isems = (isem0, isem1)
    ssems = (ssem0, ssem1)

    def idx_off(g):
        return ebase + jnp.minimum(g, NCHUNK - 1) * CH

    def start_idx(g, k):
        off = idx_off(g)
        pltpu.async_copy(src_hbm.at[pl.ds(off, CH)], srcs[k], isems[k])
        pltpu.async_copy(dst_hbm.at[pl.ds(off, CH)], dsts[k], isems[k])

    def wait_idx(g, k):
        off = idx_off(g)
        pltpu.make_async_copy(src_hbm.at[pl.ds(off, CH)], srcs[k],
                              isems[k]).wait()
        pltpu.make_async_copy(dst_hbm.at[pl.ds(off, CH)], dsts[k],
                              isems[k]).wait()

    start_idx(0, 0)
    start_idx(1, 1)

    def body(t, carry):
        g0 = 2 * t
        g1 = g0 + 1
        wait_idx(g0, 0)
        pltpu.async_copy(tab_sh.at[srcs[0]], rows[0], gsem).wait()
        h0 = pltpu.async_copy(rows[0], acc_sh.at[dsts[0]], ssems[0],
                              add=True)
        wait_idx(g1, 1)
        pltpu.async_copy(tab_sh.at[srcs[1]], rows[1], gsem).wait()
        h0.wait()
        start_idx(g0 + 2, 0)
        h1 = pltpu.async_copy(rows[1], acc_sh.at[dsts[1]], ssems[1],
                              add=True)
        h1.wait()
        start_idx(g1 + 2, 1)
        return carry

    lax.fori_loop(0, NCHUNK // 2, body, 0)
    wait_idx(NCHUNK, 0)
    wait_idx(NCHUNK + 1, 1)
    plsc.subcore_barrier()

    def drain(j, carry):
        r = rb + j * SRPT
        pltpu.sync_copy(acc_sh.at[pl.ds(r, SRPT)], stage_v)
        pltpu.sync_copy(stage_v, out_hbm.at[c].at[pl.ds(r, SRPT)])
        return carry

    lax.fori_loop(0, 8, drain, 0)


_segsum = functools.partial(
    pl.kernel,
    out_type=jax.ShapeDtypeStruct((NC, NP, W), jnp.float32),
    mesh=plsc.VectorSubcoreMesh(core_axis_name="c", subcore_axis_name="s"),
    compiler_params=pltpu.CompilerParams(use_tc_tiling_on_sc=False),
    scratch_types=[
        pltpu.VMEM_SHARED((NP, W), jnp.float32),
        pltpu.VMEM_SHARED((NP, W), jnp.float32),
        pltpu.VMEM((SRPT, W), jnp.float32),
        pltpu.VMEM((CH,), jnp.int32),
        pltpu.VMEM((CH,), jnp.int32),
        pltpu.VMEM((CH,), jnp.int32),
        pltpu.VMEM((CH,), jnp.int32),
        pltpu.VMEM((CH, W), jnp.float32),
        pltpu.VMEM((CH, W), jnp.float32),
        pltpu.SemaphoreType.DMA,
        pltpu.SemaphoreType.DMA,
        pltpu.SemaphoreType.DMA,
        pltpu.SemaphoreType.DMA,
        pltpu.SemaphoreType.DMA,
    ],
)(_segsum_body)


def _final_body(tab, parts, invm, ssum, cnod,
                aW2, ab2, aW3, ab3, aW4c, ab4, cW2, cb2, cW3, cb3, cW4c, cb4,
                pol_ref, val_ref, lscr, mscr, escr, xscr):
    ph = pl.program_id(0)
    b = pl.program_id(1)

    @pl.when(ph == 0)
    def _mlp():
        @pl.when(b == 0)
        def _init():
            mscr[0, 0] = -1e30
            xscr[...] = jnp.zeros((1, W), jnp.float32)

        acc = parts[0] + parts[1]
        deg = jnp.maximum(acc[:, 5:6], 1.0)
        x8 = acc / deg
        iv = invm[...]
        featn = tab[...] * iv
        xn = x8 * iv
        h = (featn + xn) * 0.5
        a1 = jax.nn.sigmoid(jnp.dot(h, aW2[...],
                                    preferred_element_type=jnp.float32)
                            + ab2[...])
        a2 = jax.nn.sigmoid(jnp.dot(a1, aW3[...],
                                    preferred_element_type=jnp.float32)
                            + ab3[...])
        logit = jnp.sum(a2 * aW4c[...], axis=1) + ab4[0, 0]
        ridx = b * BLK + lax.broadcasted_iota(jnp.int32, (BLK,), 0)
        bad = (ridx >= N) | (ridx == cnod[0, 0])
        logit = jnp.where(bad, -1e30, logit)
        lscr[b, :] = logit
        mscr[0, 0] = jnp.maximum(mscr[0, 0], jnp.max(logit))
        xscr[...] = xscr[...] + jnp.sum(x8, axis=0, keepdims=True)

    @pl.when(ph == 1)
    def _out():
        m = mscr[0, 0]

        @pl.when(b == 0)
        def _sum():
            def sbody(j, acc_s):
                return acc_s + jnp.sum(jnp.exp(lscr[j, :] - m))
            escr[0, 0] = lax.fori_loop(0, NBLK, sbody, 0.0)

        pol_ref[...] = jnp.exp(lscr[b, :] - m) * (1.0 / escr[0, 0])

        @pl.when(b == NBLK - 1)
        def _critic():
            iv = invm[...]
            s_ = ssum[...] * iv
            xs = xscr[...] * iv
            rn = 1.0 / float(N)
            cc = jnp.concatenate(
                [s_[:, 0:1], s_[:, 1:2], s_[:, 3:4],
                 xs[:, 0:1], xs[:, 1:2], xs[:, 3:4]], axis=1) * rn
            v1 = jax.nn.sigmoid(jnp.dot(cc, cW2[...],
                                        preferred_element_type=jnp.float32)
                                + cb2[...])
            v2 = jax.nn.sigmoid(jnp.dot(v1, cW3[...],
                                        preferred_element_type=jnp.float32)
                                + cb3[...])
            val_ref[0, 0] = jnp.sum(v2 * cW4c[...]) + cb4[0, 0]


def _final(tab, parts, invm, ssum, cnod,
           aW2p, ab2, aW3, ab3, aW4c, ab4, cW2, cb2, cW3, cb3, cW4c, cb4):
    reps = lambda shape: pl.BlockSpec(shape, lambda p, b: tuple(0 for _ in shape))
    return pl.pallas_call(
        _final_body,
        grid=(2, NBLK),
        in_specs=[
            pl.BlockSpec((BLK, W), lambda p, b: (b, 0)),
            pl.BlockSpec((NC, BLK, W), lambda p, b: (0, b, 0)),
            reps((1, W)),
            reps((1, W)),
            pl.BlockSpec(memory_space=pltpu.SMEM),
            reps((W, 256)),
            reps((1, 256)),
            reps((256, 128)),
            reps((1, 128)),
            reps((1, 128)),
            pl.BlockSpec(memory_space=pltpu.SMEM),
            reps((6, 256)),
            reps((1, 256)),
            reps((256, 128)),
            reps((1, 128)),
            reps((1, 128)),
            pl.BlockSpec(memory_space=pltpu.SMEM),
        ],
        out_specs=[
            pl.BlockSpec((BLK,), lambda p, b: (b,)),
            pl.BlockSpec(memory_space=pltpu.SMEM),
        ],
        out_shape=[
            jax.ShapeDtypeStruct((NP,), jnp.float32),
            jax.ShapeDtypeStruct((1, 1), jnp.float32),
        ],
        scratch_shapes=[
            pltpu.VMEM((NBLK, BLK), jnp.float32),
            pltpu.SMEM((1, 1), jnp.float32),
            pltpu.SMEM((1, 1), jnp.float32),
            pltpu.VMEM((1, W), jnp.float32),
        ],
    )(tab, parts, invm, ssum, cnod,
      aW2p, ab2, aW3, ab3, aW4c, ab4, cW2, cb2, cW3, cb3, cW4c, cb4)


def kernel(edge_index, hop, remaining_distance, expected_travel_time,
           current_time, curr2neigh, curr2target, nid, curr_node,
           aW2, ab2, aW3, ab3, aW4, ab4, cW2, cb2, cW3, cb3, cW4, cb4):
    tab, pmax, psum = _prep(hop, remaining_distance, expected_travel_time,
                            current_time, curr2neigh, curr2target)

    m = jnp.max(pmax[:, 0, :], axis=0)
    ssum = jnp.sum(psum[:, 0, :], axis=0, keepdims=True)
    colw = jnp.arange(W)
    norm_col = (colw >= 1) & (colw <= 4)
    invm = jnp.where(norm_col, 1.0 / jnp.maximum(m, 1e-12), 1.0)
    invm = invm.reshape(1, W).astype(jnp.float32)

    src = edge_index[0]
    dst = edge_index[1]
    zeros = jnp.zeros((NP, W), jnp.float32)
    parts = _segsum(tab, zeros, src, dst)

    aW2p = jnp.pad(aW2, ((0, 3), (0, 0)))
    cnod = jnp.asarray(curr_node, jnp.int32).reshape(1, 1)
    pol_pad, val = _final(
        tab, parts, invm, ssum, cnod,
        aW2p, ab2.reshape(1, 256), aW3, ab3.reshape(1, 128),
        aW4.reshape(1, 128), ab4.reshape(1, 1).astype(jnp.float32),
        cW2, cb2.reshape(1, 256), cW3, cb3.reshape(1, 128),
        cW4.reshape(1, 128), cb4.reshape(1, 1).astype(jnp.float32))

    policy_mask = pol_pad[:N]
    value = val.reshape(())
    return policy_mask, value
